# baseline (speedup 1.0000x reference)
import functools

import jax
import jax.numpy as jnp
from jax import lax
from jax.experimental import pallas as pl
from jax.experimental.pallas import tpu as pltpu
from jax.experimental.pallas import tpu_sc as plsc

EMBED_DIM = 64
B_TOTAL = 16384 * 26
NC, NS = 2, 16
NW = NC * NS
B_PER_W = B_TOTAL // NW
CHUNK = 512
N_CHUNKS = B_PER_W // CHUNK

_MESH = plsc.VectorSubcoreMesh(core_axis_name="c", subcore_axis_name="s")


@functools.partial(
    pl.kernel,
    mesh=_MESH,
    compiler_params=pltpu.CompilerParams(use_tc_tiling_on_sc=False),
    out_type=jax.ShapeDtypeStruct((B_TOTAL, EMBED_DIM), jnp.float32),
    scratch_types=[
        pltpu.VMEM((CHUNK,), jnp.int32),
        pltpu.VMEM((CHUNK, EMBED_DIM), jnp.float32),
        pltpu.SemaphoreType.DMA,
    ],
)
def _embed_gather(idx_hbm, table_hbm, out_hbm, idx_v, rows_v, sem):
    wid = lax.axis_index("s") * NC + lax.axis_index("c")
    base = wid * B_PER_W

    def chunk_body(i, carry):
        off = base + i * CHUNK
        pltpu.sync_copy(idx_hbm.at[pl.ds(off, CHUNK)], idx_v)
        pltpu.async_copy(table_hbm.at[idx_v], rows_v, sem).wait()
        pltpu.sync_copy(rows_v, out_hbm.at[pl.ds(off, CHUNK)])
        return carry

    lax.fori_loop(0, N_CHUNKS, chunk_body, 0)


def kernel(embed_input, weight):
    idx = embed_input.reshape(-1).astype(jnp.int32)
    out = _embed_gather(idx, weight)
    return out.reshape(embed_input.shape + (EMBED_DIM,))


# speedup vs baseline: 1.0306x; 1.0306x over previous
import functools

import jax
import jax.numpy as jnp
from jax import lax
from jax.experimental import pallas as pl
from jax.experimental.pallas import tpu as pltpu
from jax.experimental.pallas import tpu_sc as plsc

EMBED_DIM = 64
B_TOTAL = 16384 * 26
NC, NS = 2, 16
NW = NC * NS
B_PER_W = B_TOTAL // NW
NBUF = 4
CHUNK = 416
N_CHUNKS = B_PER_W // CHUNK

_MESH = plsc.VectorSubcoreMesh(core_axis_name="c", subcore_axis_name="s")


@functools.partial(
    pl.kernel,
    mesh=_MESH,
    compiler_params=pltpu.CompilerParams(use_tc_tiling_on_sc=False),
    out_type=jax.ShapeDtypeStruct((B_TOTAL, EMBED_DIM), jnp.float32),
    scratch_types=[
        pltpu.VMEM((B_PER_W,), jnp.int32),
        pltpu.VMEM((NBUF, CHUNK, EMBED_DIM), jnp.float32),
    ]
    + [pltpu.SemaphoreType.DMA] * (2 * NBUF),
)
def _embed_gather(idx_hbm, table_hbm, out_hbm, idx_v, rows_v, *sems):
    gsems, osems = sems[:NBUF], sems[NBUF:]
    wid = lax.axis_index("s") * NC + lax.axis_index("c")
    base = wid * B_PER_W

    pltpu.sync_copy(idx_hbm.at[pl.ds(base, B_PER_W)], idx_v)

    def gather_copy(i, b):
        return pltpu.make_async_copy(
            table_hbm.at[idx_v.at[pl.ds(i * CHUNK, CHUNK)]], rows_v.at[b],
            gsems[b])

    def out_copy(i, b):
        return pltpu.make_async_copy(
            rows_v.at[b], out_hbm.at[pl.ds(base + i * CHUNK, CHUNK)],
            osems[b])

    for b in range(NBUF):
        gather_copy(b, b).start()

    def outer(j, carry):
        for b in range(NBUF):
            i = j * NBUF + b
            bp = (b + NBUF - 1) % NBUF

            @pl.when(jnp.logical_and(i >= 1, i <= N_CHUNKS - NBUF))
            def _():
                out_copy(i - 1, bp).wait()
                gather_copy(i - 1 + NBUF, bp).start()

            gather_copy(i, b).wait()
            out_copy(i, b).start()
        return carry

    lax.fori_loop(0, N_CHUNKS // NBUF, outer, 0)

    for b in range(NBUF):
        out_copy(N_CHUNKS - NBUF + b, b).wait()


def kernel(embed_input, weight):
    idx = embed_input.reshape(-1).astype(jnp.int32)
    out = _embed_gather(idx, weight)
    return out.reshape(embed_input.shape + (EMBED_DIM,))


# speedup vs baseline: 1.3894x; 1.3481x over previous
import functools

import jax
import jax.numpy as jnp
from jax import lax
from jax.experimental import pallas as pl
from jax.experimental.pallas import tpu as pltpu
from jax.experimental.pallas import tpu_sc as plsc

BATCH = 16384
FIELDS = 26
EMBED_DIM = 64
B_TOTAL = BATCH * FIELDS
NC, NS = 2, 16
NW = NC * NS
B_PER_W = B_TOTAL // NW
BATCH_PER_W = BATCH // NW
CHUNK_B = 16
CHUNK = CHUNK_B * FIELDS
N_CHUNKS = BATCH_PER_W // CHUNK_B
LANES = 16

_MESH = plsc.VectorSubcoreMesh(core_axis_name="c", subcore_axis_name="s")


@functools.partial(
    pl.kernel,
    mesh=_MESH,
    out_type=jax.ShapeDtypeStruct((BATCH, FIELDS, EMBED_DIM), jnp.float32),
    scratch_types=[
        pltpu.VMEM((B_PER_W,), jnp.int32),
        pltpu.VMEM((2, CHUNK, EMBED_DIM), jnp.float32),
        pltpu.SemaphoreType.DMA,
        pltpu.SemaphoreType.DMA,
        pltpu.SemaphoreType.DMA,
        pltpu.SemaphoreType.DMA,
    ],
)
def _embed_gather(idx_hbm, table_hbm, out_hbm, idx_v, rows_v, g0, g1, o0, o1):
    gsems = (g0, g1)
    osems = (o0, o1)
    wid = lax.axis_index("s") * NC + lax.axis_index("c")
    base = wid * B_PER_W
    batch_base = wid * BATCH_PER_W

    pltpu.sync_copy(idx_hbm.at[pl.ds(base, B_PER_W)], idx_v)

    def gather_start(i, b):
        def group(g, carry):
            vec = idx_v[pl.ds(i * CHUNK + g * LANES, LANES)]
            for l in range(LANES):
                r = vec[l]
                k = g * LANES + l
                pltpu.make_async_copy(
                    table_hbm.at[r], rows_v.at[b].at[k], gsems[b]).start()
            return carry
        lax.fori_loop(0, CHUNK // LANES, group, 0)

    def gather_wait(b):
        pltpu.make_async_copy(
            table_hbm.at[pl.ds(0, CHUNK)], rows_v.at[b], gsems[b]).wait()

    def out_copy(i, b):
        b0 = batch_base + i * CHUNK_B
        return [
            pltpu.make_async_copy(
                rows_v.at[b].at[pl.ds(k * FIELDS, FIELDS)],
                out_hbm.at[b0 + k], osems[b])
            for k in range(CHUNK_B)
        ]

    def out_start(i, b):
        for c in out_copy(i, b):
            c.start()

    def out_wait(i, b):
        for c in out_copy(i, b):
            c.wait()

    gather_start(0, 0)
    gather_start(1, 1)

    def step(i, carry):
        b = lax.rem(i, 2)
        del b
        return carry

    def outer(j, carry):
        for b in range(2):
            i = j * 2 + b
            bp = 1 - b

            @pl.when(jnp.logical_and(i >= 1, i <= N_CHUNKS - 2))
            def _():
                out_wait(i - 1, bp)
                gather_start(i + 1, bp)

            gather_wait(b)
            out_start(i, b)
        return carry

    lax.fori_loop(0, N_CHUNKS // 2, outer, 0)

    for b in range(2):
        out_wait(N_CHUNKS - 2 + b, b)


def kernel(embed_input, weight):
    idx = embed_input.reshape(-1).astype(jnp.int32)
    return _embed_gather(idx, weight)
